# Initial kernel scaffold; baseline (speedup 1.0000x reference)
import jax
import jax.numpy as jnp
from jax.experimental import pallas as pl


def kernel(x, offsets, W_embed, W_lin, b_lin, W_ffm, W1, b1, g1, be1, W2, b2, g2, be2, W3, b3):
    raise NotImplementedError("write your pallas kernel here")


# speedup vs baseline: 1.8259x; 1.8259x over previous
import functools

import jax
import jax.numpy as jnp
import numpy as np
from jax import lax
from jax.experimental import pallas as pl
from jax.experimental.pallas import tpu as pltpu
from jax.experimental.pallas import tpu_sc as plsc

B = 4096
F = 26
D = 16
FIELD = 3846
V = F * FIELD
H1, H2 = 256, 128
EOD = F * D

NC, NS = 2, 16
NW = NC * NS
SPW = B // NW
XPW = SPW * F

NPAIR = (F * (F - 1)) // 2
NSLOT = 2 * NPAIR
NSLOT_PAD = 656
FFM_CHUNKS = [(0, 128), (128, 128), (256, 128), (384, 128), (512, 128), (640, 16)]


def _pair_tables():
    cols, bases = [], []
    for i in range(F - 1):
        for j in range(i + 1, F):
            cols.append(i)
            bases.append(j * V)
            cols.append(j)
            bases.append(i * V)
    while len(cols) < NSLOT_PAD:
        cols.append(0)
        bases.append(0)
    return np.asarray(cols, np.int32), np.asarray(bases, np.int32)


_PCOL_NP, _PBASE_NP = _pair_tables()


def _sc_body(xi_hbm, wffm_hbm, wemb_hbm, wlin_hbm, pcol_hbm, pbase_hbm,
             out_sc, out_emb,
             xi_v, pcol_v, pbase_v, idx_v, rows_v, emb_v, lin_v, out_v,
             sem, sem2):
    wid = lax.axis_index("s") * NC + lax.axis_index("c")
    sbase = wid * SPW
    xbase = wid * XPW

    pltpu.sync_copy(xi_hbm.at[pl.ds(xbase, XPW)], xi_v)
    pltpu.sync_copy(pcol_hbm, pcol_v)
    pltpu.sync_copy(pbase_hbm, pbase_v)

    handles = []
    for k in range(F):
        handles.append(pltpu.async_copy(
            wemb_hbm.at[xi_v.at[pl.ds(k * 128, 128)]],
            emb_v.at[pl.ds(k * 128, 128)], sem2))
    for k in range(F):
        handles.append(pltpu.async_copy(
            wlin_hbm.at[xi_v.at[pl.ds(k * 128, 128)]],
            lin_v.at[pl.ds(k * 128, 128)], sem2))
    for h in handles:
        h.wait()
    pltpu.sync_copy(emb_v, out_emb.at[pl.ds(xbase, XPW)])

    iota = lax.iota(jnp.int32, 16)

    def body(s, acc):
        for c in range(NSLOT_PAD // 16):
            cols = pcol_v[pl.ds(c * 16, 16)]
            bases = pbase_v[pl.ds(c * 16, 16)]
            xiv = plsc.load_gather(xi_v, [s * F + cols])
            idx_v[pl.ds(c * 16, 16)] = xiv + bases
        hs = []
        for off, ln in FFM_CHUNKS:
            hs.append(pltpu.async_copy(
                wffm_hbm.at[idx_v.at[pl.ds(off, ln)]],
                rows_v.at[pl.ds(off, ln)], sem))
        for h in hs:
            h.wait()
        a0 = rows_v[0] * rows_v[1]
        a1 = rows_v[2] * rows_v[3]
        a2 = rows_v[4] * rows_v[5]
        a3 = rows_v[6] * rows_v[7]
        for k in range(4, NPAIR, 4):
            a0 = a0 + rows_v[2 * k] * rows_v[2 * k + 1]
            if k + 1 < NPAIR:
                a1 = a1 + rows_v[2 * k + 2] * rows_v[2 * k + 3]
            if k + 2 < NPAIR:
                a2 = a2 + rows_v[2 * k + 4] * rows_v[2 * k + 5]
            if k + 3 < NPAIR:
                a3 = a3 + rows_v[2 * k + 6] * rows_v[2 * k + 7]
        ffm = jnp.sum((a0 + a1) + (a2 + a3))
        v0 = plsc.load_gather(lin_v, [s * F + iota])
        i1 = jnp.where(iota < F - 16, s * F + 16 + iota, 0)
        v1 = jnp.where(iota < F - 16, plsc.load_gather(lin_v, [i1]),
                       jnp.zeros((16,), jnp.float32))
        val = ffm + jnp.sum(v0 + v1)
        lane = lax.rem(s, 16)
        acc = jnp.where(iota == lane, val, acc)

        @pl.when(lane == 15)
        def _():
            plsc.store_scatter(out_v, [(s - 15) + iota], acc)

        return acc

    lax.fori_loop(0, SPW, body, jnp.zeros((16,), jnp.float32))
    pltpu.sync_copy(out_v, out_sc.at[pl.ds(sbase, SPW)])


_sc_call = functools.partial(
    pl.kernel,
    out_type=(
        jax.ShapeDtypeStruct((B,), jnp.float32),
        jax.ShapeDtypeStruct((B * F, D), jnp.float32),
    ),
    mesh=plsc.VectorSubcoreMesh(core_axis_name="c", subcore_axis_name="s"),
    scratch_types=[
        pltpu.VMEM((XPW,), jnp.int32),
        pltpu.VMEM((NSLOT_PAD,), jnp.int32),
        pltpu.VMEM((NSLOT_PAD,), jnp.int32),
        pltpu.VMEM((NSLOT_PAD,), jnp.int32),
        pltpu.VMEM((NSLOT_PAD, D), jnp.float32),
        pltpu.VMEM((XPW, D), jnp.float32),
        pltpu.VMEM((XPW,), jnp.float32),
        pltpu.VMEM((SPW,), jnp.float32),
        pltpu.SemaphoreType.DMA,
        pltpu.SemaphoreType.DMA,
    ],
    compiler_params=pltpu.CompilerParams(
        needs_layout_passes=False, use_tc_tiling_on_sc=False),
)(_sc_body)


def _mlp_body(h_ref, w1, b1, g1, be1, w2, b2, g2, be2, w3, b3, blin, sc_ref,
              out_ref):
    eps = 1e-5
    z = jnp.dot(h_ref[...], w1[...], preferred_element_type=jnp.float32)
    z = z + b1[...][None, :]
    mu = jnp.mean(z, axis=0, keepdims=True)
    var = jnp.mean((z - mu) ** 2, axis=0, keepdims=True)
    h1 = jnp.maximum((z - mu) / jnp.sqrt(var + eps) * g1[...][None, :]
                     + be1[...][None, :], 0.0)
    z2 = jnp.dot(h1, w2[...], preferred_element_type=jnp.float32)
    z2 = z2 + b2[...][None, :]
    mu2 = jnp.mean(z2, axis=0, keepdims=True)
    var2 = jnp.mean((z2 - mu2) ** 2, axis=0, keepdims=True)
    h2 = jnp.maximum((z2 - mu2) / jnp.sqrt(var2 + eps) * g2[...][None, :]
                     + be2[...][None, :], 0.0)
    mlp = jnp.dot(h2, w3[...], preferred_element_type=jnp.float32)[:, 0]
    out_ref[...] = jax.nn.sigmoid(sc_ref[...] + mlp + b3[0] + blin[0])


_mlp_call = pl.pallas_call(
    _mlp_body,
    out_shape=jax.ShapeDtypeStruct((B,), jnp.float32),
)


def kernel(x, offsets, W_embed, W_lin, b_lin, W_ffm, W1, b1, g1, be1,
           W2, b2, g2, be2, W3, b3):
    xi = (x + offsets[None, :]).reshape(B * F)
    wffm2 = W_ffm.reshape(F * V, D)
    wlin1 = W_lin.reshape(V)
    pcol = jnp.asarray(_PCOL_NP)
    pbase = jnp.asarray(_PBASE_NP)
    sc_out, emb = _sc_call(xi, wffm2, W_embed, wlin1, pcol, pbase)
    h = emb.reshape(B, EOD)
    return _mlp_call(h, W1, b1, g1, be1, W2, b2, g2, be2, W3, b3, b_lin,
                     sc_out)
